# baseline (speedup 1.0000x reference)
import dataclasses
import functools

import jax
import jax.numpy as jnp
from jax import lax
from jax.experimental import pallas as pl
from jax.experimental.pallas import tpu as pltpu
from jax.experimental.pallas import tpu_sc as plsc


def _sc_compiler_params(tc_tiling=True):
    cp = pltpu.CompilerParams()
    if "needs_layout_passes" in pltpu.CompilerParams.__dataclass_fields__:
        cp = dataclasses.replace(cp, needs_layout_passes=False)
    if not tc_tiling:
        cp = dataclasses.replace(cp, use_tc_tiling_on_sc=False)
    return cp


NC = 2
NS = 16
NW = NC * NS
LANES = 128


def _make_deg_kernel(N, E):
    EPW = E // NW
    HR = (N + LANES - 1) // LANES
    mesh = plsc.VectorSubcoreMesh(core_axis_name="c", subcore_axis_name="s")

    @functools.partial(
        pl.kernel,
        out_type=jax.ShapeDtypeStruct((2 * NW, HR, LANES), jnp.float32),
        mesh=mesh,
        scratch_types=[
            pltpu.VMEM((EPW,), jnp.int32),
            pltpu.VMEM((EPW,), jnp.int32),
            pltpu.VMEM((HR, LANES), jnp.float32),
            pltpu.VMEM((HR, LANES), jnp.float32),
        ],
        compiler_params=_sc_compiler_params(),
    )
    def deg_kernel(edge_hbm, out_hbm, src_v, dst_v, hs_v, hd_v):
        c = lax.axis_index("c")
        s = lax.axis_index("s")
        wid = s * NC + c
        base = wid * EPW
        pltpu.sync_copy(edge_hbm.at[pl.ds(base, EPW)], src_v)
        pltpu.sync_copy(edge_hbm.at[pl.ds(E + base, EPW)], dst_v)

        zeros16 = jnp.zeros((16,), jnp.float32)

        @pl.loop(0, HR)
        def _(i):
            @pl.loop(0, LANES, step=16)
            def _(j):
                hs_v[i, pl.ds(j, 16)] = zeros16
                hd_v[i, pl.ds(j, 16)] = zeros16

        ones16 = jnp.ones((16,), jnp.float32)

        @pl.loop(0, EPW, step=16)
        def _(i):
            sv = src_v[pl.ds(i, 16)]
            dv = dst_v[pl.ds(i, 16)]
            plsc.addupdate_scatter(
                hs_v,
                [lax.shift_right_logical(sv, 7), lax.bitwise_and(sv, 127)],
                ones16,
            )
            plsc.addupdate_scatter(
                hd_v,
                [lax.shift_right_logical(dv, 7), lax.bitwise_and(dv, 127)],
                ones16,
            )

        pltpu.sync_copy(hs_v, out_hbm.at[wid])
        pltpu.sync_copy(hd_v, out_hbm.at[NW + wid])

    return deg_kernel


def _make_agg_kernel(N, E, D, K=80, NB=2):
    EPW = E // NW
    NCH = EPW // K
    HR = (N + LANES - 1) // LANES
    NPAD = HR * LANES
    NPT = NPAD // NS
    RD = 2 * NB
    mesh = plsc.VectorSubcoreMesh(core_axis_name="c", subcore_axis_name="s")

    @functools.partial(
        pl.kernel,
        out_type=jax.ShapeDtypeStruct((NC * NPAD, D), jnp.float32),
        mesh=mesh,
        scratch_types=[
            pltpu.VMEM((RD, K), jnp.int32),
            pltpu.VMEM((RD, K), jnp.int32),
            [pltpu.VMEM((K, D), jnp.float32)] * NB,
            pltpu.VMEM((8, D), jnp.float32),
            pltpu.VMEM_SHARED((NPAD, D), jnp.float32),
            [pltpu.SemaphoreType.DMA] * RD,
            [pltpu.SemaphoreType.DMA] * NB,
        ],
        compiler_params=_sc_compiler_params(tc_tiling=(D % LANES == 0)),
    )
    def agg_kernel(h_hbm, edge_hbm, out_hbm,
                   sring, dring, rows, z_v, acc_sp, isem, gsem):
        c = lax.axis_index("c")
        s = lax.axis_index("s")
        wid = s * NC + c
        base = wid * EPW

        def idx_issue(cc, j):
            pltpu.async_copy(edge_hbm.at[pl.ds(base + cc * K, K)],
                             sring.at[j], isem[j])
            pltpu.async_copy(edge_hbm.at[pl.ds(E + base + cc * K, K)],
                             dring.at[j], isem[j])

        def idx_wait(j):
            pltpu.make_async_copy(edge_hbm.at[pl.ds(base, K)],
                                  sring.at[j], isem[j]).wait()
            pltpu.make_async_copy(edge_hbm.at[pl.ds(E + base, K)],
                                  dring.at[j], isem[j]).wait()

        def gather_issue(j, b):
            pltpu.async_copy(h_hbm.at[sring.at[j]], rows[b], gsem[b])

        def gather_wait(j, b):
            pltpu.make_async_copy(h_hbm.at[sring.at[j]], rows[b],
                                  gsem[b]).wait()

        for j in range(RD):
            idx_issue(j, j)

        zeros16 = jnp.zeros((16,), jnp.float32)

        @pl.loop(0, 8)
        def _(i):
            @pl.loop(0, D, step=16)
            def _(j):
                z_v[i, pl.ds(j, 16)] = zeros16

        @pl.loop(0, NPT, step=8)
        def _(r):
            pltpu.sync_copy(z_v, acc_sp.at[pl.ds(s * NPT + r, 8)])

        for b in range(NB):
            idx_wait(b)
            gather_issue(b, b)

        plsc.subcore_barrier()

        @pl.loop(0, NCH, step=RD)
        def _(ch):
            for j in range(RD):
                cc = ch + j
                b = j % NB

                @pl.when(cc < NCH)
                def _():
                    gather_wait(j, b)
                    pltpu.sync_copy(rows[b], acc_sp.at[dring.at[j]], add=True)

                    @pl.when(cc + RD < NCH)
                    def _():
                        idx_issue(cc + RD, j)

                    @pl.when(cc + NB < NCH)
                    def _():
                        idx_wait((j + NB) % RD)
                        gather_issue((j + NB) % RD, b)

        plsc.subcore_barrier()
        pltpu.sync_copy(acc_sp.at[pl.ds(s * NPT, NPT)],
                        out_hbm.at[pl.ds(c * NPAD + s * NPT, NPT)])

    return agg_kernel


def _make_degsum_body(HR):
    def degsum_body(dp_ref, ro_ref, ri_ref):
        so = jnp.sum(dp_ref[:NW], axis=0)
        si = jnp.sum(dp_ref[NW:], axis=0)
        ro = lax.rsqrt(jnp.maximum(so, 1.0))
        ri = lax.rsqrt(jnp.maximum(si, 1.0))
        for hi in range(HR):
            ro_ref[pl.ds(hi * LANES, LANES), :] = jnp.transpose(
                ro[hi:hi + 1, :])
            ri_ref[pl.ds(hi * LANES, LANES), :] = jnp.transpose(
                ri[hi:hi + 1, :])
    return degsum_body


def _lin1_body(x_ref, w_ref, ro_ref, h_ref):
    h_ref[...] = jnp.dot(x_ref[...] * ro_ref[...], w_ref[...],
                         preferred_element_type=jnp.float32)


def _mid_body(a0_ref, a1_ref, ri_ref, ro_ref, b1_ref, w2_ref, h2_ref):
    a = a0_ref[...] + a1_ref[...]
    h1 = (a * ri_ref[...] + b1_ref[...][None, :]) * ro_ref[...]
    h2_ref[...] = jnp.dot(h1, w2_ref[...], preferred_element_type=jnp.float32)


def _out_body(a0_ref, a1_ref, ri_ref, b2_ref, o_ref):
    a = a0_ref[...] + a1_ref[...]
    o_ref[...] = a * ri_ref[...] + b2_ref[...][None, :]


def kernel(features, edge_index, W1, b1, W2, b2):
    N, D_IN = features.shape
    E = edge_index.shape[1]
    D_H = W1.shape[1]
    D_OUT = W2.shape[1]
    HR = (N + LANES - 1) // LANES
    NPAD = HR * LANES

    edge_flat = edge_index.reshape(2 * E)

    degpart = _make_deg_kernel(N, E)(edge_flat)
    ro, ri = pl.pallas_call(
        _make_degsum_body(HR),
        out_shape=(jax.ShapeDtypeStruct((NPAD, 1), jnp.float32),
                   jax.ShapeDtypeStruct((NPAD, 1), jnp.float32)),
    )(degpart)

    GB = NPAD // NS
    row_spec = lambda d: pl.BlockSpec((GB, d), lambda i: (i, 0))
    p0_spec = lambda d: pl.BlockSpec((GB, d), lambda i: (i, 0))
    p1_spec = lambda d: pl.BlockSpec((GB, d), lambda i: (NS + i, 0))
    whole2 = lambda r, c: pl.BlockSpec((r, c), lambda i: (0, 0))
    whole1 = lambda d: pl.BlockSpec((d,), lambda i: (0,))

    h = pl.pallas_call(
        _lin1_body,
        grid=(NS,),
        in_specs=[row_spec(D_IN), whole2(D_IN, D_H), row_spec(1)],
        out_specs=row_spec(D_H),
        out_shape=jax.ShapeDtypeStruct((N, D_H), jnp.float32),
    )(features, W1, ro)

    agg1 = _make_agg_kernel(N, E, D_H, NB=4)(h, edge_flat)

    h2 = pl.pallas_call(
        _mid_body,
        grid=(NS,),
        in_specs=[p0_spec(D_H), p1_spec(D_H), row_spec(1), row_spec(1),
                  whole1(D_H), whole2(D_H, D_OUT)],
        out_specs=row_spec(D_OUT),
        out_shape=jax.ShapeDtypeStruct((N, D_OUT), jnp.float32),
    )(agg1, agg1, ri, ro, b1, W2)

    agg2 = _make_agg_kernel(N, E, D_OUT, NB=6)(h2, edge_flat)

    out = pl.pallas_call(
        _out_body,
        grid=(NS,),
        in_specs=[p0_spec(D_OUT), p1_spec(D_OUT), row_spec(1), whole1(D_OUT)],
        out_specs=row_spec(D_OUT),
        out_shape=jax.ShapeDtypeStruct((N, D_OUT), jnp.float32),
    )(agg2, agg2, ri, b2)

    return out
